# baseline (speedup 1.0000x reference)
import functools

import jax
import jax.numpy as jnp
from jax import lax
from jax.experimental import pallas as pl
from jax.experimental.pallas import tpu as pltpu
from jax.experimental.pallas import tpu_sc as plsc

_VOCAB = 1000000
_DIM = 64
_BATCH = 4096
_NUM_SAMPLED = 8192

_NC = 2
_NS = 16
_NW = _NC * _NS

_TRUE_PER_W = _BATCH // _NW
_SAMP_PER_W = _NUM_SAMPLED // _NW


_TC_LOG = 12
_TCOLS = 1 << _TC_LOG
_THALF = _TCOLS // 2
_TGRID = -(-_VOCAB // _TCOLS)
_PK_ROWS = _TGRID * _THALF


def _transpose_body(wt_ref, out_ref):
  x = wt_ref[...]
  xt = jnp.transpose(x)
  out_ref[:, 0:_DIM] = xt[0:_THALF]
  out_ref[:, _DIM:2 * _DIM] = xt[_THALF:]


def _tc_transpose(weights):
  wt = weights.T
  return pl.pallas_call(
      _transpose_body,
      grid=(_TGRID,),
      in_specs=[pl.BlockSpec((_DIM, _TCOLS), lambda j: (0, j))],
      out_specs=pl.BlockSpec((_THALF, 2 * _DIM), lambda j: (j, 0)),
      out_shape=jax.ShapeDtypeStruct((_PK_ROWS, 2 * _DIM), jnp.float32),
  )(wt)


def _packed_row(r):
  return lax.bitwise_or(
      lax.shift_left(lax.shift_right_logical(r, _TC_LOG), _TC_LOG - 1),
      lax.bitwise_and(r, _THALF - 1))


def _sc_gather(w_pk, biases, true_classes, sampled):
  mesh = plsc.VectorSubcoreMesh(core_axis_name="c", subcore_axis_name="s")

  @functools.partial(
      pl.kernel,
      out_type=[
          jax.ShapeDtypeStruct((_BATCH, 2 * _DIM), jnp.float32),
          jax.ShapeDtypeStruct((_BATCH,), jnp.float32),
          jax.ShapeDtypeStruct((_NUM_SAMPLED, 2 * _DIM), jnp.float32),
          jax.ShapeDtypeStruct((_NUM_SAMPLED,), jnp.float32),
      ],
      mesh=mesh,
      compiler_params=pltpu.CompilerParams(skip_device_barrier=True),
      scratch_types=[
          pltpu.VMEM((_TRUE_PER_W,), jnp.int32),
          pltpu.VMEM((_TRUE_PER_W,), jnp.int32),
          pltpu.VMEM((_TRUE_PER_W, 2 * _DIM), jnp.float32),
          pltpu.VMEM((_TRUE_PER_W,), jnp.float32),
          pltpu.VMEM((_SAMP_PER_W,), jnp.int32),
          pltpu.VMEM((_SAMP_PER_W,), jnp.int32),
          pltpu.VMEM((_SAMP_PER_W, 2 * _DIM), jnp.float32),
          pltpu.VMEM((_SAMP_PER_W,), jnp.float32),
          pltpu.SemaphoreType.DMA,
          pltpu.SemaphoreType.DMA,
          pltpu.SemaphoreType.DMA,
          pltpu.SemaphoreType.DMA,
      ],
  )
  def gather(w_hbm, b_hbm, tc_hbm, s_hbm,
             tw_out, tb_out, sw_out, sb_out,
             tidx_v, tgidx_v, trow_v, tb_v,
             sidx_v, sgidx_v, srow_v, sb_v,
             sem0, sem1, sem2, sem3):
    wid = lax.axis_index("s") * _NC + lax.axis_index("c")
    tbase = wid * _TRUE_PER_W
    sbase = wid * _SAMP_PER_W
    pltpu.sync_copy(tc_hbm.at[pl.ds(tbase, _TRUE_PER_W)], tidx_v)
    pltpu.sync_copy(s_hbm.at[pl.ds(sbase, _SAMP_PER_W)], sidx_v)
    for k in range(_TRUE_PER_W // 16):
      tgidx_v[pl.ds(16 * k, 16)] = _packed_row(tidx_v[pl.ds(16 * k, 16)])
    for k in range(_SAMP_PER_W // 16):
      sgidx_v[pl.ds(16 * k, 16)] = _packed_row(sidx_v[pl.ds(16 * k, 16)])
    c0 = pltpu.async_copy(w_hbm.at[tgidx_v], trow_v, sem0)
    c1 = pltpu.async_copy(w_hbm.at[sgidx_v], srow_v, sem1)
    c2 = pltpu.async_copy(b_hbm.at[tidx_v], tb_v, sem2)
    c3 = pltpu.async_copy(b_hbm.at[sidx_v], sb_v, sem3)
    c0.wait()
    c1.wait()
    c2.wait()
    c3.wait()
    pltpu.sync_copy(trow_v, tw_out.at[pl.ds(tbase, _TRUE_PER_W)])
    pltpu.sync_copy(tb_v, tb_out.at[pl.ds(tbase, _TRUE_PER_W)])
    pltpu.sync_copy(srow_v, sw_out.at[pl.ds(sbase, _SAMP_PER_W)])
    pltpu.sync_copy(sb_v, sb_out.at[pl.ds(sbase, _SAMP_PER_W)])

  return gather(w_pk, biases, true_classes, sampled)


_BB = 256
_SS = 2048


def _sel_half(wide, ids_col):
  half = lax.bitwise_and(lax.shift_right_logical(ids_col, _TC_LOG - 1), 1)
  return jnp.where(half == 1, wide[:, _DIM:2 * _DIM], wide[:, 0:_DIM])


_LOG2E = 1.4426950408889634
_LN2 = 0.6931471805599453


def _softplus(z):
  return _LN2 * jnp.log2(1.0 + jnp.exp2(z * _LOG2E))


def _nce_body(x_ref, tw_ref, tb_ref, tc_ref, tcc_ref, sw_ref, sb_ref,
              sid_ref, sidc_ref, out_ref, wsel_ref, csb_ref):
  i = pl.program_id(0)
  j = pl.program_id(1)

  @pl.when(i == 0)
  def _():
    wsel_ref[pl.ds(j * _SS, _SS), :] = _sel_half(
        sw_ref[...], sidc_ref[...]).astype(jnp.bfloat16)
    sid = sid_ref[...].astype(jnp.float32)
    q = (jnp.log(sid + 2.0) - jnp.log(sid + 1.0)) / jnp.log(_VOCAB + 1.0)
    csb_ref[:, pl.ds(j * _SS, _SS)] = (
        sb_ref[...] - jnp.log(_NUM_SAMPLED * q + 1e-12))

  x = x_ref[...]
  logits = lax.dot_general(
      x.astype(jnp.bfloat16), wsel_ref[pl.ds(j * _SS, _SS), :],
      dimension_numbers=(((1,), (1,)), ((), ())),
      preferred_element_type=jnp.float32)
  part = jnp.sum(_softplus(logits + csb_ref[:, pl.ds(j * _SS, _SS)]),
                 axis=1)

  @pl.when(j == 0)
  def _():
    tcid = tc_ref[0, :].astype(jnp.float32)
    qt = (jnp.log(tcid + 2.0) - jnp.log(tcid + 1.0)) / jnp.log(_VOCAB + 1.0)
    tw = _sel_half(tw_ref[...], tcc_ref[...])
    tl = (jnp.sum(x * tw, axis=1) + tb_ref[0, :]
          - jnp.log(_NUM_SAMPLED * qt + 1e-12))
    out_ref[0, :] = _softplus(-tl) + part

  @pl.when(j > 0)
  def _():
    out_ref[0, :] += part


def _tc_loss(inputs, true_w, true_b, true_classes, sampled_w, sampled_b,
             sampled):
  grid = (_BATCH // _BB, _NUM_SAMPLED // _SS)
  out = pl.pallas_call(
      _nce_body,
      grid=grid,
      in_specs=[
          pl.BlockSpec((_BB, _DIM), lambda i, j: (i, 0)),
          pl.BlockSpec((_BB, 2 * _DIM), lambda i, j: (i, 0)),
          pl.BlockSpec((1, _BB), lambda i, j: (0, i)),
          pl.BlockSpec((1, _BB), lambda i, j: (0, i)),
          pl.BlockSpec((_BB, 1), lambda i, j: (i, 0)),
          pl.BlockSpec((_SS, 2 * _DIM), lambda i, j: (j, 0)),
          pl.BlockSpec((1, _SS), lambda i, j: (0, j)),
          pl.BlockSpec((1, _SS), lambda i, j: (0, j)),
          pl.BlockSpec((_SS, 1), lambda i, j: (j, 0)),
      ],
      out_specs=pl.BlockSpec((1, _BB), lambda i, j: (0, i)),
      out_shape=jax.ShapeDtypeStruct((1, _BATCH), jnp.float32),
      scratch_shapes=[
          pltpu.VMEM((_NUM_SAMPLED, _DIM), jnp.bfloat16),
          pltpu.VMEM((1, _NUM_SAMPLED), jnp.float32),
      ],
      compiler_params=pltpu.CompilerParams(
          dimension_semantics=("arbitrary", "arbitrary")),
  )(inputs, true_w, true_b.reshape(1, _BATCH),
    true_classes.reshape(1, _BATCH), true_classes.reshape(_BATCH, 1),
    sampled_w, sampled_b.reshape(1, _NUM_SAMPLED),
    sampled.reshape(1, _NUM_SAMPLED), sampled.reshape(_NUM_SAMPLED, 1))
  return out.reshape(_BATCH)


def kernel(inputs, true_classes, sampled, weights, biases):
  w_pk = _tc_transpose(weights)
  true_w, true_b, sampled_w, sampled_b = _sc_gather(
      w_pk, biases, true_classes, sampled)
  return _tc_loss(inputs, true_w, true_b, true_classes, sampled_w,
                  sampled_b, sampled)


# speedup vs baseline: 1.4156x; 1.4156x over previous
import functools

import jax
import jax.numpy as jnp
from jax import lax
from jax.experimental import pallas as pl
from jax.experimental.pallas import tpu as pltpu
from jax.experimental.pallas import tpu_sc as plsc

_VOCAB = 1000000
_DIM = 64
_BATCH = 4096
_NUM_SAMPLED = 8192

_NC = 2
_NS = 16
_NW = _NC * _NS

_TRUE_PER_W = _BATCH // _NW
_SAMP_PER_W = _NUM_SAMPLED // _NW


_TC_LOG = 13
_TCOLS = 1 << _TC_LOG
_TQ = _TCOLS // 4
_TGRID = -(-_VOCAB // _TCOLS)
_PK_ROWS = _TGRID * _TQ

_HI_MASK = -65536


def _pack_pair(a, b):
  ia = lax.bitcast_convert_type(
      a.astype(jnp.bfloat16).astype(jnp.float32), jnp.int32)
  ib = lax.bitcast_convert_type(
      b.astype(jnp.bfloat16).astype(jnp.float32), jnp.int32)
  return lax.bitcast_convert_type(
      lax.bitwise_or(ia, lax.shift_right_logical(ib, 16)), jnp.float32)


def _transpose_body(wt_ref, out_ref):
  x = wt_ref[...]
  xt = jnp.transpose(x)
  out_ref[:, 0:_DIM] = _pack_pair(xt[0:_TQ], xt[_TQ:2 * _TQ])
  out_ref[:, _DIM:2 * _DIM] = _pack_pair(xt[2 * _TQ:3 * _TQ], xt[3 * _TQ:])


def _tc_transpose(weights):
  wt = weights.T
  return pl.pallas_call(
      _transpose_body,
      grid=(_TGRID,),
      in_specs=[pl.BlockSpec((_DIM, _TCOLS), lambda j: (0, j))],
      out_specs=pl.BlockSpec((_TQ, 2 * _DIM), lambda j: (j, 0)),
      out_shape=jax.ShapeDtypeStruct((_PK_ROWS, 2 * _DIM), jnp.float32),
  )(wt)


def _packed_row(r):
  return lax.bitwise_or(
      lax.shift_left(lax.shift_right_logical(r, _TC_LOG), _TC_LOG - 2),
      lax.bitwise_and(r, _TQ - 1))


def _sc_gather(w_pk, biases, true_classes, sampled):
  mesh = plsc.VectorSubcoreMesh(core_axis_name="c", subcore_axis_name="s")

  @functools.partial(
      pl.kernel,
      out_type=[
          jax.ShapeDtypeStruct((_BATCH, 2 * _DIM), jnp.float32),
          jax.ShapeDtypeStruct((_BATCH,), jnp.float32),
          jax.ShapeDtypeStruct((_NUM_SAMPLED, 2 * _DIM), jnp.float32),
          jax.ShapeDtypeStruct((_NUM_SAMPLED,), jnp.float32),
      ],
      mesh=mesh,
      compiler_params=pltpu.CompilerParams(skip_device_barrier=True),
      scratch_types=[
          pltpu.VMEM((_TRUE_PER_W,), jnp.int32),
          pltpu.VMEM((_TRUE_PER_W,), jnp.int32),
          pltpu.VMEM((_TRUE_PER_W, 2 * _DIM), jnp.float32),
          pltpu.VMEM((_TRUE_PER_W,), jnp.float32),
          pltpu.VMEM((_SAMP_PER_W,), jnp.int32),
          pltpu.VMEM((_SAMP_PER_W,), jnp.int32),
          pltpu.VMEM((_SAMP_PER_W, 2 * _DIM), jnp.float32),
          pltpu.VMEM((_SAMP_PER_W,), jnp.float32),
          pltpu.SemaphoreType.DMA,
          pltpu.SemaphoreType.DMA,
          pltpu.SemaphoreType.DMA,
          pltpu.SemaphoreType.DMA,
      ],
  )
  def gather(w_hbm, b_hbm, tc_hbm, s_hbm,
             tw_out, tb_out, sw_out, sb_out,
             tidx_v, tgidx_v, trow_v, tb_v,
             sidx_v, sgidx_v, srow_v, sb_v,
             sem0, sem1, sem2, sem3):
    wid = lax.axis_index("s") * _NC + lax.axis_index("c")
    tbase = wid * _TRUE_PER_W
    sbase = wid * _SAMP_PER_W
    pltpu.sync_copy(tc_hbm.at[pl.ds(tbase, _TRUE_PER_W)], tidx_v)
    pltpu.sync_copy(s_hbm.at[pl.ds(sbase, _SAMP_PER_W)], sidx_v)
    for k in range(_TRUE_PER_W // 16):
      tgidx_v[pl.ds(16 * k, 16)] = _packed_row(tidx_v[pl.ds(16 * k, 16)])
    for k in range(_SAMP_PER_W // 16):
      sgidx_v[pl.ds(16 * k, 16)] = _packed_row(sidx_v[pl.ds(16 * k, 16)])
    c0 = pltpu.async_copy(w_hbm.at[tgidx_v], trow_v, sem0)
    c1 = pltpu.async_copy(w_hbm.at[sgidx_v], srow_v, sem1)
    c2 = pltpu.async_copy(b_hbm.at[tidx_v], tb_v, sem2)
    c3 = pltpu.async_copy(b_hbm.at[sidx_v], sb_v, sem3)
    c0.wait()
    c1.wait()
    c2.wait()
    c3.wait()
    pltpu.sync_copy(trow_v, tw_out.at[pl.ds(tbase, _TRUE_PER_W)])
    pltpu.sync_copy(tb_v, tb_out.at[pl.ds(tbase, _TRUE_PER_W)])
    pltpu.sync_copy(srow_v, sw_out.at[pl.ds(sbase, _SAMP_PER_W)])
    pltpu.sync_copy(sb_v, sb_out.at[pl.ds(sbase, _SAMP_PER_W)])

  return gather(w_pk, biases, true_classes, sampled)


_BB = 256
_SS = 2048


def _sel_half(wide, ids_col):
  lane_half = lax.bitwise_and(
      lax.shift_right_logical(ids_col, _TC_LOG - 1), 1)
  top_bot = lax.bitwise_and(
      lax.shift_right_logical(ids_col, _TC_LOG - 2), 1)
  w32 = jnp.where(lane_half == 1, wide[:, _DIM:2 * _DIM], wide[:, 0:_DIM])
  bits = lax.bitcast_convert_type(w32, jnp.int32)
  sel = jnp.where(top_bot == 1, lax.shift_left(bits, 16),
                  lax.bitwise_and(bits, jnp.int32(_HI_MASK)))
  return lax.bitcast_convert_type(sel, jnp.float32)


_LOG2E = 1.4426950408889634
_LN2 = 0.6931471805599453


def _softplus(z):
  return _LN2 * jnp.log2(1.0 + jnp.exp2(z * _LOG2E))


def _nce_body(x_ref, tw_ref, tb_ref, tc_ref, tcc_ref, sw_ref, sb_ref,
              sid_ref, sidc_ref, out_ref, wsel_ref, csb_ref):
  i = pl.program_id(0)
  j = pl.program_id(1)

  @pl.when(i == 0)
  def _():
    wsel_ref[pl.ds(j * _SS, _SS), :] = _sel_half(
        sw_ref[...], sidc_ref[...]).astype(jnp.bfloat16)
    sid = sid_ref[...].astype(jnp.float32)
    q = (jnp.log(sid + 2.0) - jnp.log(sid + 1.0)) / jnp.log(_VOCAB + 1.0)
    csb_ref[:, pl.ds(j * _SS, _SS)] = (
        sb_ref[...] - jnp.log(_NUM_SAMPLED * q + 1e-12))

  x = x_ref[...]
  logits = lax.dot_general(
      x.astype(jnp.bfloat16), wsel_ref[pl.ds(j * _SS, _SS), :],
      dimension_numbers=(((1,), (1,)), ((), ())),
      preferred_element_type=jnp.float32)
  part = jnp.sum(_softplus(logits + csb_ref[:, pl.ds(j * _SS, _SS)]),
                 axis=1)

  @pl.when(j == 0)
  def _():
    tcid = tc_ref[0, :].astype(jnp.float32)
    qt = (jnp.log(tcid + 2.0) - jnp.log(tcid + 1.0)) / jnp.log(_VOCAB + 1.0)
    tw = _sel_half(tw_ref[...], tcc_ref[...])
    tl = (jnp.sum(x * tw, axis=1) + tb_ref[0, :]
          - jnp.log(_NUM_SAMPLED * qt + 1e-12))
    out_ref[0, :] = _softplus(-tl) + part

  @pl.when(j > 0)
  def _():
    out_ref[0, :] += part


def _tc_loss(inputs, true_w, true_b, true_classes, sampled_w, sampled_b,
             sampled):
  grid = (_BATCH // _BB, _NUM_SAMPLED // _SS)
  out = pl.pallas_call(
      _nce_body,
      grid=grid,
      in_specs=[
          pl.BlockSpec((_BB, _DIM), lambda i, j: (i, 0)),
          pl.BlockSpec((_BB, 2 * _DIM), lambda i, j: (i, 0)),
          pl.BlockSpec((1, _BB), lambda i, j: (0, i)),
          pl.BlockSpec((1, _BB), lambda i, j: (0, i)),
          pl.BlockSpec((_BB, 1), lambda i, j: (i, 0)),
          pl.BlockSpec((_SS, 2 * _DIM), lambda i, j: (j, 0)),
          pl.BlockSpec((1, _SS), lambda i, j: (0, j)),
          pl.BlockSpec((1, _SS), lambda i, j: (0, j)),
          pl.BlockSpec((_SS, 1), lambda i, j: (j, 0)),
      ],
      out_specs=pl.BlockSpec((1, _BB), lambda i, j: (0, i)),
      out_shape=jax.ShapeDtypeStruct((1, _BATCH), jnp.float32),
      scratch_shapes=[
          pltpu.VMEM((_NUM_SAMPLED, _DIM), jnp.bfloat16),
          pltpu.VMEM((1, _NUM_SAMPLED), jnp.float32),
      ],
      compiler_params=pltpu.CompilerParams(
          dimension_semantics=("arbitrary", "arbitrary")),
  )(inputs, true_w, true_b.reshape(1, _BATCH),
    true_classes.reshape(1, _BATCH), true_classes.reshape(_BATCH, 1),
    sampled_w, sampled_b.reshape(1, _NUM_SAMPLED),
    sampled.reshape(1, _NUM_SAMPLED), sampled.reshape(_NUM_SAMPLED, 1))
  return out.reshape(_BATCH)


def kernel(inputs, true_classes, sampled, weights, biases):
  w_pk = _tc_transpose(weights)
  true_w, true_b, sampled_w, sampled_b = _sc_gather(
      w_pk, biases, true_classes, sampled)
  return _tc_loss(inputs, true_w, true_b, true_classes, sampled_w,
                  sampled_b, sampled)


# speedup vs baseline: 1.5796x; 1.1158x over previous
import functools

import jax
import jax.numpy as jnp
from jax import lax
from jax.experimental import pallas as pl
from jax.experimental.pallas import tpu as pltpu
from jax.experimental.pallas import tpu_sc as plsc

_VOCAB = 1000000
_DIM = 64
_BATCH = 4096
_NUM_SAMPLED = 8192

_NC = 2
_NS = 16
_NW = _NC * _NS

_TRUE_PER_W = _BATCH // _NW
_SAMP_PER_W = _NUM_SAMPLED // _NW


_TC_LOG = 13
_TCOLS = 1 << _TC_LOG
_TQ = _TCOLS // 4
_TGRID = -(-_VOCAB // _TCOLS)
_PK_ROWS = _TGRID * _TQ

_HI_MASK = -65536


def _pack_pair(a, b):
  ia = lax.bitcast_convert_type(
      a.astype(jnp.bfloat16).astype(jnp.float32), jnp.int32)
  ib = lax.bitcast_convert_type(
      b.astype(jnp.bfloat16).astype(jnp.float32), jnp.int32)
  return lax.bitcast_convert_type(
      lax.bitwise_or(ia, lax.shift_right_logical(ib, 16)), jnp.float32)


def _transpose_body(wt_ref, out_ref):
  x = wt_ref[...]
  xt = jnp.transpose(x)
  out_ref[:, 0:_DIM] = _pack_pair(xt[0:_TQ], xt[_TQ:2 * _TQ])
  out_ref[:, _DIM:2 * _DIM] = _pack_pair(xt[2 * _TQ:3 * _TQ], xt[3 * _TQ:])


def _tc_transpose(weights):
  wt = weights.T
  return pl.pallas_call(
      _transpose_body,
      grid=(_TGRID,),
      in_specs=[pl.BlockSpec((_DIM, _TCOLS), lambda j: (0, j))],
      out_specs=pl.BlockSpec((_TQ, 2 * _DIM), lambda j: (j, 0)),
      out_shape=jax.ShapeDtypeStruct((_PK_ROWS, 2 * _DIM), jnp.float32),
  )(wt)


def _packed_row(r):
  return lax.bitwise_or(
      lax.shift_left(lax.shift_right_logical(r, _TC_LOG), _TC_LOG - 2),
      lax.bitwise_and(r, _TQ - 1))


def _sc_gather(w_pk, biases, true_classes, sampled):
  mesh = plsc.VectorSubcoreMesh(core_axis_name="c", subcore_axis_name="s")

  @functools.partial(
      pl.kernel,
      out_type=[
          jax.ShapeDtypeStruct((_BATCH, 2 * _DIM), jnp.float32),
          jax.ShapeDtypeStruct((_BATCH,), jnp.float32),
          jax.ShapeDtypeStruct((_NUM_SAMPLED, 2 * _DIM), jnp.float32),
          jax.ShapeDtypeStruct((_NUM_SAMPLED,), jnp.float32),
      ],
      mesh=mesh,
      compiler_params=pltpu.CompilerParams(skip_device_barrier=True),
      scratch_types=[
          pltpu.VMEM((_TRUE_PER_W,), jnp.int32),
          pltpu.VMEM((_TRUE_PER_W,), jnp.int32),
          pltpu.VMEM((_TRUE_PER_W, 2 * _DIM), jnp.float32),
          pltpu.VMEM((_TRUE_PER_W,), jnp.float32),
          pltpu.VMEM((_SAMP_PER_W,), jnp.int32),
          pltpu.VMEM((_SAMP_PER_W,), jnp.int32),
          pltpu.VMEM((_SAMP_PER_W, 2 * _DIM), jnp.float32),
          pltpu.VMEM((_SAMP_PER_W,), jnp.float32),
          pltpu.SemaphoreType.DMA,
          pltpu.SemaphoreType.DMA,
          pltpu.SemaphoreType.DMA,
          pltpu.SemaphoreType.DMA,
      ],
  )
  def gather(w_hbm, b_hbm, tc_hbm, s_hbm,
             tw_out, tb_out, sw_out, sb_out,
             tidx_v, tgidx_v, trow_v, tb_v,
             sidx_v, sgidx_v, srow_v, sb_v,
             sem0, sem1, sem2, sem3):
    wid = lax.axis_index("s") * _NC + lax.axis_index("c")
    tbase = wid * _TRUE_PER_W
    sbase = wid * _SAMP_PER_W
    pltpu.sync_copy(tc_hbm.at[pl.ds(tbase, _TRUE_PER_W)], tidx_v)
    pltpu.sync_copy(s_hbm.at[pl.ds(sbase, _SAMP_PER_W)], sidx_v)
    for k in range(_TRUE_PER_W // 16):
      tgidx_v[pl.ds(16 * k, 16)] = _packed_row(tidx_v[pl.ds(16 * k, 16)])
    for k in range(_SAMP_PER_W // 16):
      sgidx_v[pl.ds(16 * k, 16)] = _packed_row(sidx_v[pl.ds(16 * k, 16)])
    c0 = pltpu.async_copy(w_hbm.at[tgidx_v], trow_v, sem0)
    c1 = pltpu.async_copy(w_hbm.at[sgidx_v], srow_v, sem1)
    c2 = pltpu.async_copy(b_hbm.at[tidx_v], tb_v, sem2)
    c3 = pltpu.async_copy(b_hbm.at[sidx_v], sb_v, sem3)
    c0.wait()
    c1.wait()
    c2.wait()
    c3.wait()
    pltpu.sync_copy(trow_v, tw_out.at[pl.ds(tbase, _TRUE_PER_W)])
    pltpu.sync_copy(tb_v, tb_out.at[pl.ds(tbase, _TRUE_PER_W)])
    pltpu.sync_copy(srow_v, sw_out.at[pl.ds(sbase, _SAMP_PER_W)])
    pltpu.sync_copy(sb_v, sb_out.at[pl.ds(sbase, _SAMP_PER_W)])

  return gather(w_pk, biases, true_classes, sampled)


_BB = 512
_SS = 2048


def _sel_half(wide, ids_col):
  lane_half = lax.bitwise_and(
      lax.shift_right_logical(ids_col, _TC_LOG - 1), 1)
  top_bot = lax.bitwise_and(
      lax.shift_right_logical(ids_col, _TC_LOG - 2), 1)
  w32 = jnp.where(lane_half == 1, wide[:, _DIM:2 * _DIM], wide[:, 0:_DIM])
  bits = lax.bitcast_convert_type(w32, jnp.int32)
  sel = jnp.where(top_bot == 1, lax.shift_left(bits, 16),
                  lax.bitwise_and(bits, jnp.int32(_HI_MASK)))
  return lax.bitcast_convert_type(sel, jnp.float32)


_LOG2E = 1.4426950408889634
_LN2 = 0.6931471805599453


def _softplus(z):
  return _LN2 * jnp.log2(1.0 + jnp.exp2(z * _LOG2E))


def _nce_body(x_ref, tw_ref, tbc_ref, tcc_ref, sw_ref, sb_ref,
              sid_ref, sidc_ref, out_ref, wsel_ref, csb_ref, xb_ref):
  i = pl.program_id(0)
  j = pl.program_id(1)

  @pl.when(i == 0)
  def _():
    wsel_ref[pl.ds(j * _SS, _SS), :] = _sel_half(
        sw_ref[...], sidc_ref[...]).astype(jnp.bfloat16)
    sid = sid_ref[...].astype(jnp.float32)
    q = (jnp.log(sid + 2.0) - jnp.log(sid + 1.0)) / jnp.log(_VOCAB + 1.0)
    csb_ref[:, pl.ds(j * _SS, _SS)] = (
        sb_ref[...] - jnp.log(_NUM_SAMPLED * q + 1e-12))

  @pl.when(j == 0)
  def _():
    xb_ref[...] = x_ref[...].astype(jnp.bfloat16)

  logits = lax.dot_general(
      xb_ref[...], wsel_ref[pl.ds(j * _SS, _SS), :],
      dimension_numbers=(((1,), (1,)), ((), ())),
      preferred_element_type=jnp.float32)
  sp = _softplus(logits + csb_ref[:, pl.ds(j * _SS, _SS)])
  ones = jnp.ones((_SS, 1), jnp.float32)
  part = lax.dot_general(sp, ones, (((1,), (0,)), ((), ())),
                         preferred_element_type=jnp.float32)

  @pl.when(j == 0)
  def _():
    tcid = tcc_ref[...].astype(jnp.float32)
    qt = (jnp.log(tcid + 2.0) - jnp.log(tcid + 1.0)) / jnp.log(_VOCAB + 1.0)
    tw = _sel_half(tw_ref[...], tcc_ref[...])
    tl = (jnp.sum(x_ref[...] * tw, axis=1, keepdims=True) + tbc_ref[...]
          - jnp.log(_NUM_SAMPLED * qt + 1e-12))
    out_ref[...] = _softplus(-tl) + part

  @pl.when(j > 0)
  def _():
    out_ref[...] += part


def _tc_loss(inputs, true_w, true_b, true_classes, sampled_w, sampled_b,
             sampled):
  grid = (_BATCH // _BB, _NUM_SAMPLED // _SS)
  out = pl.pallas_call(
      _nce_body,
      grid=grid,
      in_specs=[
          pl.BlockSpec((_BB, _DIM), lambda i, j: (i, 0)),
          pl.BlockSpec((_BB, 2 * _DIM), lambda i, j: (i, 0)),
          pl.BlockSpec((_BB, 1), lambda i, j: (i, 0)),
          pl.BlockSpec((_BB, 1), lambda i, j: (i, 0)),
          pl.BlockSpec((_SS, 2 * _DIM), lambda i, j: (j, 0)),
          pl.BlockSpec((1, _SS), lambda i, j: (0, j)),
          pl.BlockSpec((1, _SS), lambda i, j: (0, j)),
          pl.BlockSpec((_SS, 1), lambda i, j: (j, 0)),
      ],
      out_specs=pl.BlockSpec((_BB, 1), lambda i, j: (i, 0)),
      out_shape=jax.ShapeDtypeStruct((_BATCH, 1), jnp.float32),
      scratch_shapes=[
          pltpu.VMEM((_NUM_SAMPLED, _DIM), jnp.bfloat16),
          pltpu.VMEM((1, _NUM_SAMPLED), jnp.float32),
          pltpu.VMEM((_BB, _DIM), jnp.bfloat16),
      ],
      compiler_params=pltpu.CompilerParams(
          dimension_semantics=("arbitrary", "arbitrary")),
  )(inputs, true_w, true_b.reshape(_BATCH, 1),
    true_classes.reshape(_BATCH, 1),
    sampled_w, sampled_b.reshape(1, _NUM_SAMPLED),
    sampled.reshape(1, _NUM_SAMPLED), sampled.reshape(_NUM_SAMPLED, 1))
  return out.reshape(_BATCH)


def kernel(inputs, true_classes, sampled, weights, biases):
  w_pk = _tc_transpose(weights)
  true_w, true_b, sampled_w, sampled_b = _sc_gather(
      w_pk, biases, true_classes, sampled)
  return _tc_loss(inputs, true_w, true_b, true_classes, sampled_w,
                  sampled_b, sampled)


# speedup vs baseline: 1.8144x; 1.1487x over previous
import functools

import jax
import jax.numpy as jnp
from jax import lax
from jax.experimental import pallas as pl
from jax.experimental.pallas import tpu as pltpu
from jax.experimental.pallas import tpu_sc as plsc

_VOCAB = 1000000
_DIM = 64
_BATCH = 4096
_NUM_SAMPLED = 8192

_NC = 2
_NS = 16
_NW = _NC * _NS

_TRUE_PER_W = _BATCH // _NW
_SAMP_PER_W = _NUM_SAMPLED // _NW


_TC_LOG = 14
_TCOLS = 1 << _TC_LOG
_TQ = _TCOLS // 4
_TGRID = -(-_VOCAB // _TCOLS)
_PK_ROWS = _TGRID * _TQ

_HI_MASK = -65536


def _pack_pair(a, b):
  ia = lax.bitcast_convert_type(
      a.astype(jnp.bfloat16).astype(jnp.float32), jnp.int32)
  ib = lax.bitcast_convert_type(
      b.astype(jnp.bfloat16).astype(jnp.float32), jnp.int32)
  return lax.bitcast_convert_type(
      lax.bitwise_or(ia, lax.shift_right_logical(ib, 16)), jnp.float32)


def _transpose_body(wt_ref, out_ref):
  x = wt_ref[...]
  xt = jnp.transpose(x)
  out_ref[:, 0:_DIM] = _pack_pair(xt[0:_TQ], xt[_TQ:2 * _TQ])
  out_ref[:, _DIM:2 * _DIM] = _pack_pair(xt[2 * _TQ:3 * _TQ], xt[3 * _TQ:])


def _tc_transpose(weights):
  wt = weights.T
  return pl.pallas_call(
      _transpose_body,
      grid=(_TGRID,),
      in_specs=[pl.BlockSpec((_DIM, _TCOLS), lambda j: (0, j))],
      out_specs=pl.BlockSpec((_TQ, 2 * _DIM), lambda j: (j, 0)),
      out_shape=jax.ShapeDtypeStruct((_PK_ROWS, 2 * _DIM), jnp.float32),
  )(wt)


def _packed_row(r):
  return lax.bitwise_or(
      lax.shift_left(lax.shift_right_logical(r, _TC_LOG), _TC_LOG - 2),
      lax.bitwise_and(r, _TQ - 1))


def _sc_gather(w_pk, biases, true_classes, sampled):
  mesh = plsc.VectorSubcoreMesh(core_axis_name="c", subcore_axis_name="s")

  @functools.partial(
      pl.kernel,
      out_type=[
          jax.ShapeDtypeStruct((_BATCH, 2 * _DIM), jnp.float32),
          jax.ShapeDtypeStruct((_BATCH,), jnp.float32),
          jax.ShapeDtypeStruct((_NUM_SAMPLED, 2 * _DIM), jnp.float32),
          jax.ShapeDtypeStruct((_NUM_SAMPLED,), jnp.float32),
      ],
      mesh=mesh,
      compiler_params=pltpu.CompilerParams(skip_device_barrier=True),
      scratch_types=[
          pltpu.VMEM((_TRUE_PER_W,), jnp.int32),
          pltpu.VMEM((_TRUE_PER_W,), jnp.int32),
          pltpu.VMEM((_TRUE_PER_W, 2 * _DIM), jnp.float32),
          pltpu.VMEM((_TRUE_PER_W,), jnp.float32),
          pltpu.VMEM((_SAMP_PER_W,), jnp.int32),
          pltpu.VMEM((_SAMP_PER_W,), jnp.int32),
          pltpu.VMEM((_SAMP_PER_W, 2 * _DIM), jnp.float32),
          pltpu.VMEM((_SAMP_PER_W,), jnp.float32),
          pltpu.SemaphoreType.DMA,
          pltpu.SemaphoreType.DMA,
          pltpu.SemaphoreType.DMA,
          pltpu.SemaphoreType.DMA,
      ],
  )
  def gather(w_hbm, b_hbm, tc_hbm, s_hbm,
             tw_out, tb_out, sw_out, sb_out,
             tidx_v, tgidx_v, trow_v, tb_v,
             sidx_v, sgidx_v, srow_v, sb_v,
             sem0, sem1, sem2, sem3):
    wid = lax.axis_index("s") * _NC + lax.axis_index("c")
    tbase = wid * _TRUE_PER_W
    sbase = wid * _SAMP_PER_W
    pltpu.sync_copy(tc_hbm.at[pl.ds(tbase, _TRUE_PER_W)], tidx_v)
    pltpu.sync_copy(s_hbm.at[pl.ds(sbase, _SAMP_PER_W)], sidx_v)
    for k in range(_TRUE_PER_W // 16):
      tgidx_v[pl.ds(16 * k, 16)] = _packed_row(tidx_v[pl.ds(16 * k, 16)])
    for k in range(_SAMP_PER_W // 16):
      sgidx_v[pl.ds(16 * k, 16)] = _packed_row(sidx_v[pl.ds(16 * k, 16)])
    c0 = pltpu.async_copy(w_hbm.at[tgidx_v], trow_v, sem0)
    c1 = pltpu.async_copy(w_hbm.at[sgidx_v], srow_v, sem1)
    c2 = pltpu.async_copy(b_hbm.at[tidx_v], tb_v, sem2)
    c3 = pltpu.async_copy(b_hbm.at[sidx_v], sb_v, sem3)
    c0.wait()
    c1.wait()
    c2.wait()
    c3.wait()
    pltpu.sync_copy(trow_v, tw_out.at[pl.ds(tbase, _TRUE_PER_W)])
    pltpu.sync_copy(tb_v, tb_out.at[pl.ds(tbase, _TRUE_PER_W)])
    pltpu.sync_copy(srow_v, sw_out.at[pl.ds(sbase, _SAMP_PER_W)])
    pltpu.sync_copy(sb_v, sb_out.at[pl.ds(sbase, _SAMP_PER_W)])

  return gather(w_pk, biases, true_classes, sampled)


_BB = 512
_SS = 2048


def _sel_half(wide, ids_col):
  lane_half = lax.bitwise_and(
      lax.shift_right_logical(ids_col, _TC_LOG - 1), 1)
  top_bot = lax.bitwise_and(
      lax.shift_right_logical(ids_col, _TC_LOG - 2), 1)
  w32 = jnp.where(lane_half == 1, wide[:, _DIM:2 * _DIM], wide[:, 0:_DIM])
  bits = lax.bitcast_convert_type(w32, jnp.int32)
  sel = jnp.where(top_bot == 1, lax.shift_left(bits, 16),
                  lax.bitwise_and(bits, jnp.int32(_HI_MASK)))
  return lax.bitcast_convert_type(sel, jnp.float32)


_LOG2E = 1.4426950408889634
_LN2 = 0.6931471805599453


def _softplus(z):
  return _LN2 * jnp.log2(1.0 + jnp.exp2(z * _LOG2E))


def _nce_body(x_ref, tw_ref, tbc_ref, tcc_ref, sw_ref, sb_ref,
              sid_ref, sidc_ref, out_ref, wsel_ref, csb_ref, xb_ref):
  i = pl.program_id(0)
  j = pl.program_id(1)

  @pl.when(i == 0)
  def _():
    wsel_ref[pl.ds(j * _SS, _SS), :] = _sel_half(
        sw_ref[...], sidc_ref[...]).astype(jnp.bfloat16)
    sid = sid_ref[...].astype(jnp.float32)
    q = (jnp.log(sid + 2.0) - jnp.log(sid + 1.0)) / jnp.log(_VOCAB + 1.0)
    csb_ref[:, pl.ds(j * _SS, _SS)] = (
        sb_ref[...] - jnp.log(_NUM_SAMPLED * q + 1e-12))

  @pl.when(j == 0)
  def _():
    xb_ref[...] = x_ref[...].astype(jnp.bfloat16)

  logits = lax.dot_general(
      xb_ref[...], wsel_ref[pl.ds(j * _SS, _SS), :],
      dimension_numbers=(((1,), (1,)), ((), ())),
      preferred_element_type=jnp.float32)
  sp = _softplus(logits + csb_ref[:, pl.ds(j * _SS, _SS)])
  ones = jnp.ones((_SS, 1), jnp.float32)
  part = lax.dot_general(sp, ones, (((1,), (0,)), ((), ())),
                         preferred_element_type=jnp.float32)

  @pl.when(j == 0)
  def _():
    tcid = tcc_ref[...].astype(jnp.float32)
    qt = (jnp.log(tcid + 2.0) - jnp.log(tcid + 1.0)) / jnp.log(_VOCAB + 1.0)
    tw = _sel_half(tw_ref[...], tcc_ref[...])
    tl = (jnp.sum(x_ref[...] * tw, axis=1, keepdims=True) + tbc_ref[...]
          - jnp.log(_NUM_SAMPLED * qt + 1e-12))
    out_ref[...] = _softplus(-tl) + part

  @pl.when(j > 0)
  def _():
    out_ref[...] += part


def _tc_loss(inputs, true_w, true_b, true_classes, sampled_w, sampled_b,
             sampled):
  grid = (_BATCH // _BB, _NUM_SAMPLED // _SS)
  out = pl.pallas_call(
      _nce_body,
      grid=grid,
      in_specs=[
          pl.BlockSpec((_BB, _DIM), lambda i, j: (i, 0)),
          pl.BlockSpec((_BB, 2 * _DIM), lambda i, j: (i, 0)),
          pl.BlockSpec((_BB, 1), lambda i, j: (i, 0)),
          pl.BlockSpec((_BB, 1), lambda i, j: (i, 0)),
          pl.BlockSpec((_SS, 2 * _DIM), lambda i, j: (j, 0)),
          pl.BlockSpec((1, _SS), lambda i, j: (0, j)),
          pl.BlockSpec((1, _SS), lambda i, j: (0, j)),
          pl.BlockSpec((_SS, 1), lambda i, j: (j, 0)),
      ],
      out_specs=pl.BlockSpec((_BB, 1), lambda i, j: (i, 0)),
      out_shape=jax.ShapeDtypeStruct((_BATCH, 1), jnp.float32),
      scratch_shapes=[
          pltpu.VMEM((_NUM_SAMPLED, _DIM), jnp.bfloat16),
          pltpu.VMEM((1, _NUM_SAMPLED), jnp.float32),
          pltpu.VMEM((_BB, _DIM), jnp.bfloat16),
      ],
      compiler_params=pltpu.CompilerParams(
          dimension_semantics=("arbitrary", "arbitrary")),
  )(inputs, true_w, true_b.reshape(_BATCH, 1),
    true_classes.reshape(_BATCH, 1),
    sampled_w, sampled_b.reshape(1, _NUM_SAMPLED),
    sampled.reshape(1, _NUM_SAMPLED), sampled.reshape(_NUM_SAMPLED, 1))
  return out.reshape(_BATCH)


def kernel(inputs, true_classes, sampled, weights, biases):
  w_pk = _tc_transpose(weights)
  true_w, true_b, sampled_w, sampled_b = _sc_gather(
      w_pk, biases, true_classes, sampled)
  return _tc_loss(inputs, true_w, true_b, true_classes, sampled_w,
                  sampled_b, sampled)
